# Initial kernel scaffold; baseline (speedup 1.0000x reference)
import jax
import jax.numpy as jnp
from jax.experimental import pallas as pl


def kernel(inds, mask, table):
    raise NotImplementedError("write your pallas kernel here")


# speedup vs baseline: 1.1789x; 1.1789x over previous
import functools

import jax
import jax.numpy as jnp
from jax import lax
from jax.experimental import pallas as pl
from jax.experimental.pallas import tpu as pltpu
from jax.experimental.pallas import tpu_sc as plsc

B = 4096
H = 50
D = 64
HP = 64
NC = 2
NS = 16
NW = NC * NS
BPW = B // NW
CB = 2
NCHUNK = BPW // CB
CPG = 16 // CB


def _sc_body(inds_hbm, mask_hbm, table_hbm, out_hbm,
             idx_v, mask_v, rows_v, scores_v, sem):
    wid = lax.axis_index("s") * NC + lax.axis_index("c")
    pltpu.sync_copy(inds_hbm.at[pl.ds(wid * NCHUNK, NCHUNK)], idx_v)
    pltpu.sync_copy(mask_hbm.at[pl.ds(wid * BPW, BPW)], mask_v)
    lanes = lax.iota(jnp.int32, 16)

    def chunk_body(j, carry):
        dot_vec, dn_vec = carry
        pltpu.async_copy(table_hbm.at[idx_v.at[j]], rows_v, sem).wait()
        off = (j % CPG) * CB
        for t in range(CB):
            b_local = j * CB + t
            roff = t * H
            mv = [mask_v[b_local, pl.ds(16 * k, 16)] for k in range(HP // 16)]
            ms = [mv[k][i] for k in range(4) for i in range(16)]
            acc = [jnp.zeros((16,), jnp.float32) for _ in range(4)]
            dn = jnp.float32(0.0)
            for l in range(1, H):
                m = ms[l]
                dn = dn + m
                r = roff + l
                for d in range(4):
                    acc[d] = acc[d] + m * rows_v[r, pl.ds(16 * d, 16)]
            pd = jnp.zeros((16,), jnp.float32)
            for d in range(4):
                pd = pd + rows_v[roff, pl.ds(16 * d, 16)] * acc[d]
            dot = jnp.float32(0.0)
            for i in range(16):
                dot = dot + pd[i]
            here = lanes == off + t
            dot_vec = jnp.where(here, dot, dot_vec)
            dn_vec = jnp.where(here, dn, dn_vec)
        done = (j % CPG) == (CPG - 1)

        @pl.when(done)
        def _():
            scores_v[pl.ds((j // CPG) * 16, 16)] = (
                dot_vec / jnp.maximum(dn_vec, 1.0))

        z = jnp.zeros((16,), jnp.float32)
        return (jnp.where(done, z, dot_vec), jnp.where(done, z, dn_vec))

    z0 = jnp.zeros((16,), jnp.float32)
    lax.fori_loop(0, NCHUNK, chunk_body, (z0, z0))
    pltpu.sync_copy(scores_v, out_hbm.at[pl.ds(wid * BPW, BPW)])


_sc_call = functools.partial(
    pl.kernel,
    out_type=jax.ShapeDtypeStruct((B,), jnp.float32),
    mesh=plsc.VectorSubcoreMesh(core_axis_name="c", subcore_axis_name="s"),
    compiler_params=pltpu.CompilerParams(use_tc_tiling_on_sc=False),
    scratch_types=[
        pltpu.VMEM((NCHUNK, CB * H), jnp.int32),
        pltpu.VMEM((BPW, HP), jnp.float32),
        pltpu.VMEM((CB * H, D), jnp.float32),
        pltpu.VMEM((BPW,), jnp.float32),
        pltpu.SemaphoreType.DMA,
    ],
)(_sc_body)


def kernel(inds, mask, table):
    inds2 = inds.astype(jnp.int32).reshape(NW * NCHUNK, CB * H)
    mask_p = jnp.pad(mask, ((0, 0), (0, HP - H)))
    return _sc_call(inds2, mask_p, table)


# speedup vs baseline: 1.2032x; 1.0206x over previous
import functools

import jax
import jax.numpy as jnp
from jax import lax
from jax.experimental import pallas as pl
from jax.experimental.pallas import tpu as pltpu
from jax.experimental.pallas import tpu_sc as plsc

B = 4096
H = 50
D = 64
HP = 64
NC = 2
NS = 16
NW = NC * NS
BPW = B // NW
CB = 2
NCHUNK = BPW // CB
CPG = 16 // CB


NBUF = 4


def _sc_body(inds_hbm, mask_hbm, table_hbm, out_hbm,
             idx_v, mask_v, rows_b0, rows_b1, rows_b2, rows_b3, scores_v,
             sem0, sem1, sem2, sem3):
    rows_b = [rows_b0, rows_b1, rows_b2, rows_b3]
    sems = [sem0, sem1, sem2, sem3]
    wid = lax.axis_index("s") * NC + lax.axis_index("c")
    pltpu.sync_copy(inds_hbm.at[pl.ds(wid * NCHUNK, NCHUNK)], idx_v)
    pltpu.sync_copy(mask_hbm.at[pl.ds(wid * BPW, BPW)], mask_v)
    lanes = lax.iota(jnp.int32, 16)

    for b in range(NBUF):
        pltpu.async_copy(table_hbm.at[idx_v.at[b]], rows_b[b], sems[b])

    def compute_chunk(j, bb, carry):
        dot_vec, dn_vec = carry
        rows_v = rows_b[bb]
        pltpu.make_async_copy(
            table_hbm.at[idx_v.at[j]], rows_v, sems[bb]).wait()
        off = (j % CPG) * CB
        for t in range(CB):
            b_local = j * CB + t
            roff = t * H
            mv = [mask_v[b_local, pl.ds(16 * k, 16)] for k in range(HP // 16)]
            ms = [mv[k][i] for k in range(4) for i in range(16)]
            acc = [jnp.zeros((16,), jnp.float32) for _ in range(4)]
            dn = jnp.float32(0.0)
            for l in range(1, H):
                m = ms[l]
                dn = dn + m
                r = roff + l
                for d in range(4):
                    acc[d] = acc[d] + m * rows_v[r, pl.ds(16 * d, 16)]
            pd = jnp.zeros((16,), jnp.float32)
            for d in range(4):
                pd = pd + rows_v[roff, pl.ds(16 * d, 16)] * acc[d]
            dot = jnp.float32(0.0)
            for i in range(16):
                dot = dot + pd[i]
            here = lanes == off + t
            dot_vec = jnp.where(here, dot, dot_vec)
            dn_vec = jnp.where(here, dn, dn_vec)

        @pl.when(j + NBUF < NCHUNK)
        def _():
            pltpu.async_copy(
                table_hbm.at[idx_v.at[j + NBUF]], rows_v, sems[bb])

        done = (j % CPG) == (CPG - 1)

        @pl.when(done)
        def _():
            scores_v[pl.ds((j // CPG) * 16, 16)] = (
                dot_vec / jnp.maximum(dn_vec, 1.0))

        z = jnp.zeros((16,), jnp.float32)
        return (jnp.where(done, z, dot_vec), jnp.where(done, z, dn_vec))

    def ring_body(g, carry):
        for bb in range(NBUF):
            carry = compute_chunk(g * NBUF + bb, bb, carry)
        return carry

    z0 = jnp.zeros((16,), jnp.float32)
    lax.fori_loop(0, NCHUNK // NBUF, ring_body, (z0, z0))
    pltpu.sync_copy(scores_v, out_hbm.at[pl.ds(wid * BPW, BPW)])


_sc_call = functools.partial(
    pl.kernel,
    out_type=jax.ShapeDtypeStruct((B,), jnp.float32),
    mesh=plsc.VectorSubcoreMesh(core_axis_name="c", subcore_axis_name="s"),
    compiler_params=pltpu.CompilerParams(use_tc_tiling_on_sc=False),
    scratch_types=[
        pltpu.VMEM((NCHUNK, CB * H), jnp.int32),
        pltpu.VMEM((BPW, HP), jnp.float32),
        pltpu.VMEM((CB * H, D), jnp.float32),
        pltpu.VMEM((CB * H, D), jnp.float32),
        pltpu.VMEM((CB * H, D), jnp.float32),
        pltpu.VMEM((CB * H, D), jnp.float32),
        pltpu.VMEM((BPW,), jnp.float32),
        pltpu.SemaphoreType.DMA,
        pltpu.SemaphoreType.DMA,
        pltpu.SemaphoreType.DMA,
        pltpu.SemaphoreType.DMA,
    ],
)(_sc_body)


def kernel(inds, mask, table):
    inds2 = inds.astype(jnp.int32).reshape(NW * NCHUNK, CB * H)
    mask_p = jnp.pad(mask, ((0, 0), (0, HP - H)))
    return _sc_call(inds2, mask_p, table)
